# baseline (speedup 1.0000x reference)
import jax
import jax.numpy as jnp
from jax.experimental import pallas as pl
from jax.experimental.pallas import tpu as pltpu

_VMEM_LIMIT = 48 * 1024 * 1024


def _im2col9(x, H, W, C):
    cols = [x[:, kx:kx + W, :] for kx in range(3)]
    return jnp.concatenate(
        [cols[kx][ky:ky + H].reshape(H * W, C)
         for kx in range(3) for ky in range(3)],
        axis=-1)


def _halo_store(ref, interior, H, W, C):
    ref[...] = jnp.zeros((H + 2, W + 2, C), ref.dtype)
    ref[1:1 + H, 1:1 + W, :] = interior.astype(ref.dtype)


def _res_block(xp, w9, b3, w1, b1, H, W, C):
    HW = H * W
    xr = jnp.maximum(xp, 0).astype(jnp.bfloat16)
    x9 = _im2col9(xr, H, W, C)
    acc = jnp.dot(x9, w9, preferred_element_type=jnp.float32)
    h = jnp.maximum(acc + b3, 0.0).astype(jnp.bfloat16)
    out = jnp.dot(h, w1, preferred_element_type=jnp.float32) + b1
    skip = xp[1:1 + H, 1:1 + W, :].reshape(HW, C).astype(jnp.float32)
    return out + skip


def _ct_accs(xp, wct_ref, bias, relu_in, H, W, C):
    HW = H * W
    x = jnp.maximum(xp, 0) if relu_in else xp
    x = x.astype(jnp.bfloat16)
    cols = [x[:, c:c + W, :] for c in range(3)]
    s = [[cols[c][r:r + H].reshape(HW, C) for r in range(3)] for c in range(3)]
    accs = []
    for py in range(2):
        lhs = jnp.concatenate(
            [s[c][py + dy] for c in range(3) for dy in range(2)], axis=-1)
        accs.append(
            jnp.dot(lhs, wct_ref[py], preferred_element_type=jnp.float32)
            + bias)
    return accs


def _interleave_2x2(accs, H, W, C, dtype):
    rows = []
    for py in range(2):
        a = accs[py].astype(dtype)
        left = a[:, :C].reshape(H, W, C)
        right = a[:, C:].reshape(H, W, C)
        rows.append(jnp.stack([left, right], axis=2).reshape(H, 2 * W, C))
    return jnp.stack(rows, axis=1).reshape(2 * H, 2 * W, C)


def _decoder_body(B, H0, C):
    H1, H2, H3 = 2 * H0, 4 * H0, 8 * H0

    def body(x0_ref, y_ref,
             w9a_ref, b3a_ref, w1a_ref, b1a_ref,
             wct1_ref, bct1_ref,
             wc2_ref, bc2_ref,
             w9b_ref, b3b_ref, w1b_ref, b1b_ref,
             wct3_ref, bct3_ref,
             wct4_ref, bct4_ref,
             o_ref,
             m16_ref, m32a_ref, m32b_ref, m32c_ref, m64_ref):
        x1 = _res_block(x0_ref[0], w9a_ref[0], b3a_ref[0], w1a_ref[0],
                        b1a_ref[0], H0, H0, C)
        _halo_store(m16_ref, x1.reshape(H0, H0, C), H0, H0, C)
        x2 = _res_block(m16_ref[...], w9a_ref[1], b3a_ref[1], w1a_ref[1],
                        b1a_ref[1], H0, H0, C)
        _halo_store(m16_ref, x2.reshape(H0, H0, C), H0, H0, C)

        accs = _ct_accs(m16_ref[...], wct1_ref, bct1_ref[...], True,
                        H0, H0, C)
        accs = [jnp.maximum(a, 0.0) for a in accs]
        _halo_store(m32a_ref, _interleave_2x2(accs, H0, H0, C, jnp.bfloat16),
                    H1, H1, C)

        a = m32a_ref[...]
        b = y_ref[0]
        acols = [a[:, kx:kx + H1, :] for kx in range(3)]
        bcols = [b[:, kx:kx + H1, :] for kx in range(3)]
        pieces = []
        for kx in range(3):
            for ky in range(3):
                pieces.append(acols[kx][ky:ky + H1].reshape(H1 * H1, C))
                pieces.append(bcols[kx][ky:ky + H1].reshape(H1 * H1, C))
        x18 = jnp.concatenate(pieces, axis=-1)
        c2 = jnp.dot(x18, wc2_ref[...],
                     preferred_element_type=jnp.float32) + bc2_ref[...]
        _halo_store(m32b_ref, c2.reshape(H1, H1, C), H1, H1, C)

        x3 = _res_block(m32b_ref[...], w9b_ref[0], b3b_ref[0], w1b_ref[0],
                        b1b_ref[0], H1, H1, C)
        _halo_store(m32c_ref, x3.reshape(H1, H1, C), H1, H1, C)
        x4 = _res_block(m32c_ref[...], w9b_ref[1], b3b_ref[1], w1b_ref[1],
                        b1b_ref[1], H1, H1, C)
        _halo_store(m32b_ref, x4.reshape(H1, H1, C), H1, H1, C)

        accs = _ct_accs(m32b_ref[...], wct3_ref, bct3_ref[...], True,
                        H1, H1, C)
        accs = [jnp.maximum(a, 0.0) for a in accs]
        _halo_store(m64_ref, _interleave_2x2(accs, H1, H1, C, jnp.bfloat16),
                    H2, H2, C)

        accs = _ct_accs(m64_ref[...], wct4_ref, bct4_ref[...], False,
                        H2, H2, C)
        for py in range(2):
            o_ref[2 * py] = accs[py][:, :3].reshape(H2, H2, 3)
            o_ref[2 * py + 1] = accs[py][:, C:C + 3].reshape(H2, H2, 3)

    return body


def _const_spec(*shape):
    nz = len(shape)
    return pl.BlockSpec(shape, lambda b, _n=nz: (0,) * _n)


def _decoder(x0p, yp, packed):
    B, Hp, _, C = x0p.shape
    H0 = Hp - 2
    H2 = 4 * H0
    (w9a, b3a, w1a, b1a, wct1, bct1, wc2, bc2,
     w9b, b3b, w1b, b1b, wct3, bct3, wct4, bct4) = packed
    return pl.pallas_call(
        _decoder_body(B, H0, C),
        out_shape=jax.ShapeDtypeStruct((B * 4, H2, H2, 3), jnp.float32),
        grid=(B,),
        in_specs=[
            pl.BlockSpec((1, H0 + 2, H0 + 2, C), lambda b: (b, 0, 0, 0)),
            pl.BlockSpec((1, 2 * H0 + 2, 2 * H0 + 2, C),
                         lambda b: (b, 0, 0, 0)),
            _const_spec(2, 9 * C, C), _const_spec(2, 1, C),
            _const_spec(2, C, C), _const_spec(2, 1, C),
            _const_spec(2, 6 * C, 2 * C), _const_spec(1, 2 * C),
            _const_spec(18 * C, C), _const_spec(1, C),
            _const_spec(2, 9 * C, C), _const_spec(2, 1, C),
            _const_spec(2, C, C), _const_spec(2, 1, C),
            _const_spec(2, 6 * C, 2 * C), _const_spec(1, 2 * C),
            _const_spec(2, 6 * C, 2 * C), _const_spec(1, 2 * C),
        ],
        out_specs=pl.BlockSpec((4, H2, H2, 3), lambda b: (b, 0, 0, 0)),
        scratch_shapes=[
            pltpu.VMEM((H0 + 2, H0 + 2, C), jnp.bfloat16),
            pltpu.VMEM((2 * H0 + 2, 2 * H0 + 2, C), jnp.bfloat16),
            pltpu.VMEM((2 * H0 + 2, 2 * H0 + 2, C), jnp.bfloat16),
            pltpu.VMEM((2 * H0 + 2, 2 * H0 + 2, C), jnp.bfloat16),
            pltpu.VMEM((4 * H0 + 2, 4 * H0 + 2, C), jnp.bfloat16),
        ],
        compiler_params=pltpu.CompilerParams(
            dimension_semantics=("parallel",),
            vmem_limit_bytes=_VMEM_LIMIT,
        ),
    )(x0p, yp, w9a, b3a, w1a, b1a, wct1, bct1, wc2, bc2,
      w9b, b3b, w1b, b1b, wct3, bct3, wct4, bct4)


def _d2s_nchw(planar, B):
    _, H, W, C = planar.shape
    y = planar.reshape(B, 2, 2, H, W, C)
    return jnp.transpose(y, (0, 5, 3, 1, 4, 2)).reshape(B, C, 2 * H, 2 * W)


def _nchw_to_padded_nhwc(x_nchw):
    x = jnp.transpose(x_nchw, (0, 2, 3, 1))
    x = jnp.pad(x, ((0, 0), (1, 1), (1, 1), (0, 0)))
    return x.astype(jnp.bfloat16)


def _pack_w9(w3):
    return jnp.concatenate(
        [w3[:, ky * 3 + kx] for kx in range(3) for ky in range(3)], axis=1)


def _pack_cat_w(wa, wb):
    parts = []
    for kx in range(3):
        for ky in range(3):
            t = ky * 3 + kx
            parts.append(wa[t])
            parts.append(wb[t])
    return jnp.concatenate(parts, axis=0)


def _pack_ct_w(wpar):
    C, Cop = wpar.shape[-2], wpar.shape[-1]
    z = jnp.zeros((C, Cop), wpar.dtype)
    rows = []
    for py in range(2):
        kblocks = []
        for c in range(3):
            for dy in range(2):
                left = wpar[2 * py, 2 * dy + c] if c <= 1 else z
                right = wpar[2 * py + 1, 2 * dy + c - 1] if c >= 1 else z
                kblocks.append(jnp.concatenate([left, right], axis=1))
        rows.append(jnp.concatenate(kblocks, axis=0))
    return jnp.stack(rows)


def _pack_ct_b(b):
    return jnp.concatenate([b, b], axis=1)


def kernel(x0, x1, r12_w3, r12_b3, r12_w1, r12_b1,
           r34_w3, r34_b3, r34_w1, r34_b1,
           ct1_w, ct1_b, ct3_w, ct3_b, ct4_w, ct4_b,
           c2_wa, c2_wb, c2_b):
    B = x0.shape[0]
    xp = _nchw_to_padded_nhwc(x0)
    yp = _nchw_to_padded_nhwc(x1)
    packed = (
        _pack_w9(r12_w3), r12_b3, r12_w1, r12_b1,
        _pack_ct_w(ct1_w), _pack_ct_b(ct1_b),
        _pack_cat_w(c2_wa, c2_wb), c2_b,
        _pack_w9(r34_w3), r34_b3, r34_w1, r34_b1,
        _pack_ct_w(ct3_w), _pack_ct_b(ct3_b),
        _pack_ct_w(ct4_w), _pack_ct_b(ct4_b),
    )
    out = _decoder(xp, yp, packed)
    return _d2s_nchw(out, B)


# speedup vs baseline: 1.2635x; 1.2635x over previous
import jax
import jax.numpy as jnp
from jax.experimental import pallas as pl
from jax.experimental.pallas import tpu as pltpu

_VMEM_LIMIT = 48 * 1024 * 1024


def _pc(i, d):
    return i % d, i // d + 1


def _halo_store(ref, interior, dtype=None):
    h, w, c = ref.shape
    ref[...] = jnp.zeros((h, w, c), ref.dtype)
    ref[1:h - 1, 1:w - 1, :] = interior.astype(ref.dtype)


def _im2col9_single(x, H, W, C):
    cols = [x[:, kx:kx + W, :] for kx in range(3)]
    return jnp.concatenate(
        [cols[kx][ky:ky + H].reshape(H * W, C)
         for kx in range(3) for ky in range(3)],
        axis=-1)


def _plane_tap(planes, d, arow, acol, u, v, P, C):
    r, dr = _pc(arow + u - 1, d)
    c, dc = _pc(acol + v - 1, d)
    return planes[r * d + c][dr:dr + P, dc:dc + P, :].reshape(P * P, C)


def _im2col9_planes(planes, d, arow, acol, P, C):
    return jnp.concatenate(
        [_plane_tap(planes, d, arow, acol, u, v, P, C)
         for v in range(3) for u in range(3)],
        axis=-1)


def _ct_lhs(planes, d, arow, acol, py, P, C):
    pieces = []
    for c in range(3):
        cc, dc = _pc(acol + c - 1, d)
        for dy in range(2):
            rr, dr = _pc(arow + py + dy - 1, d)
            pieces.append(
                planes[rr * d + cc][dr:dr + P, dc:dc + P, :].reshape(
                    P * P, C))
    return jnp.concatenate(pieces, axis=-1)


def _res_block_single(xp, w9, b3, w1, b1, P, C):
    HW = P * P
    xr = jnp.maximum(xp, 0).astype(jnp.bfloat16)
    x9 = _im2col9_single(xr, P, P, C)
    acc = jnp.dot(x9, w9, preferred_element_type=jnp.float32)
    h = jnp.maximum(acc + b3, 0.0).astype(jnp.bfloat16)
    out = jnp.dot(h, w1, preferred_element_type=jnp.float32) + b1
    skip = xp[1:1 + P, 1:1 + P, :].reshape(HW, C).astype(jnp.float32)
    return out + skip


def _res_pair_planes(pin_refs, ptmp_refs, w9_ref, b3_ref, w1_ref, b1_ref,
                     P, C):
    HW = P * P
    n = len(pin_refs)
    d = 2 if n == 4 else 1
    for src, dst, blk in ((pin_refs, ptmp_refs, 0), (ptmp_refs, pin_refs, 1)):
        vals = [src[i][...] for i in range(n)]
        xr = [jnp.maximum(v, 0).astype(jnp.bfloat16) for v in vals]
        for a in range(d):
            for b in range(d):
                i = a * d + b
                x9 = _im2col9_planes(xr, d, a, b, P, C)
                acc = jnp.dot(x9, w9_ref[blk],
                              preferred_element_type=jnp.float32)
                h = jnp.maximum(acc + b3_ref[blk], 0.0).astype(jnp.bfloat16)
                out = jnp.dot(h, w1_ref[blk],
                              preferred_element_type=jnp.float32) + b1_ref[blk]
                skip = vals[i][1:1 + P, 1:1 + P, :].reshape(
                    HW, C).astype(jnp.float32)
                _halo_store(dst[i], (out + skip).reshape(P, P, C))


def _decoder_body(P, C):
    HW = P * P

    def body(x0_ref, y_ref,
             w9a_ref, b3a_ref, w1a_ref, b1a_ref,
             wct1_ref, bct1_ref,
             wc2_ref, bc2_ref,
             w9b_ref, b3b_ref, w1b_ref, b1b_ref,
             wct3_ref, bct3_ref,
             wct4_ref, bct4_ref,
             o_ref,
             m16_ref, pa_ref, pb_ref, pc_ref, p64_ref):
        pa = [pa_ref.at[i] for i in range(4)]
        pb = [pb_ref.at[i] for i in range(4)]
        pc = [pc_ref.at[i] for i in range(4)]
        p64 = [p64_ref.at[i] for i in range(16)]

        x1 = _res_block_single(x0_ref[0], w9a_ref[0], b3a_ref[0],
                               w1a_ref[0], b1a_ref[0], P, C)
        _halo_store(m16_ref, x1.reshape(P, P, C))
        x2 = _res_block_single(m16_ref[...], w9a_ref[1], b3a_ref[1],
                               w1a_ref[1], b1a_ref[1], P, C)
        _halo_store(m16_ref, x2.reshape(P, P, C))

        x = jnp.maximum(m16_ref[...], 0).astype(jnp.bfloat16)
        cols = [x[:, c:c + P, :] for c in range(3)]
        s = [[cols[c][r:r + P].reshape(HW, C) for r in range(3)]
             for c in range(3)]
        for py in range(2):
            lhs = jnp.concatenate(
                [s[c][py + dy] for c in range(3) for dy in range(2)], axis=-1)
            acc = jnp.dot(lhs, wct1_ref[py],
                          preferred_element_type=jnp.float32) + bct1_ref[...]
            acc = jnp.maximum(acc, 0.0)
            _halo_store(pa[2 * py], acc[:, :C].reshape(P, P, C))
            _halo_store(pa[2 * py + 1], acc[:, C:].reshape(P, P, C))

        avals = [pa_ref[i] for i in range(4)]
        yvals = [y_ref[0, i] for i in range(4)]
        for a in range(2):
            for b in range(2):
                pieces = []
                for v in range(3):
                    for u in range(3):
                        pieces.append(
                            _plane_tap(avals, 2, a, b, u, v, P, C))
                        pieces.append(
                            _plane_tap(yvals, 2, a, b, u, v, P, C))
                x18 = jnp.concatenate(pieces, axis=-1)
                out = jnp.dot(x18, wc2_ref[...],
                              preferred_element_type=jnp.float32)
                out = out + bc2_ref[...]
                _halo_store(pb[a * 2 + b], out.reshape(P, P, C))

        _res_pair_planes(pb, pc, w9b_ref, b3b_ref, w1b_ref, b1b_ref, P, C)

        xr = [jnp.maximum(pb_ref[i], 0).astype(jnp.bfloat16) for i in range(4)]
        for a in range(2):
            for b in range(2):
                for py in range(2):
                    lhs = _ct_lhs(xr, 2, a, b, py, P, C)
                    acc = jnp.dot(lhs, wct3_ref[py],
                                  preferred_element_type=jnp.float32)
                    acc = jnp.maximum(acc + bct3_ref[...], 0.0)
                    q = 2 * a + py
                    _halo_store(p64[q * 4 + 2 * b],
                                acc[:, :C].reshape(P, P, C))
                    _halo_store(p64[q * 4 + 2 * b + 1],
                                acc[:, C:].reshape(P, P, C))

        xv = [p64_ref[i] for i in range(16)]
        for q in range(4):
            for sidx in range(4):
                for py in range(2):
                    lhs = _ct_lhs(xv, 4, q, sidx, py, P, C)
                    acc = jnp.dot(lhs, wct4_ref[py],
                                  preferred_element_type=jnp.float32)
                    acc = acc + bct4_ref[...]
                    r8 = 2 * q + py
                    o_ref[0, r8, 2 * sidx] = acc[:, :3].reshape(P, P, 3)
                    o_ref[0, r8, 2 * sidx + 1] = acc[:, C:C + 3].reshape(
                        P, P, 3)

    return body


def _const_spec(*shape):
    nz = len(shape)
    return pl.BlockSpec(shape, lambda b, _n=nz: (0,) * _n)


def _decoder(x0p, yplanes, packed):
    B = x0p.shape[0]
    C = x0p.shape[-1]
    P = x0p.shape[1] - 2
    (w9a, b3a, w1a, b1a, wct1, bct1, wc2, bc2,
     w9b, b3b, w1b, b1b, wct3, bct3, wct4, bct4) = packed
    halo = P + 2
    return pl.pallas_call(
        _decoder_body(P, C),
        out_shape=jax.ShapeDtypeStruct((B, 8, 8, P, P, 3), jnp.float32),
        grid=(B,),
        in_specs=[
            pl.BlockSpec((1, halo, halo, C), lambda b: (b, 0, 0, 0)),
            pl.BlockSpec((1, 4, halo, halo, C), lambda b: (b, 0, 0, 0, 0)),
            _const_spec(2, 9 * C, C), _const_spec(2, 1, C),
            _const_spec(2, C, C), _const_spec(2, 1, C),
            _const_spec(2, 6 * C, 2 * C), _const_spec(1, 2 * C),
            _const_spec(18 * C, C), _const_spec(1, C),
            _const_spec(2, 9 * C, C), _const_spec(2, 1, C),
            _const_spec(2, C, C), _const_spec(2, 1, C),
            _const_spec(2, 6 * C, 2 * C), _const_spec(1, 2 * C),
            _const_spec(2, 6 * C, 2 * C), _const_spec(1, 2 * C),
        ],
        out_specs=pl.BlockSpec((1, 8, 8, P, P, 3),
                               lambda b: (b, 0, 0, 0, 0, 0)),
        scratch_shapes=[
            pltpu.VMEM((halo, halo, C), jnp.bfloat16),
            pltpu.VMEM((4, halo, halo, C), jnp.bfloat16),
            pltpu.VMEM((4, halo, halo, C), jnp.bfloat16),
            pltpu.VMEM((4, halo, halo, C), jnp.bfloat16),
            pltpu.VMEM((16, halo, halo, C), jnp.bfloat16),
        ],
        compiler_params=pltpu.CompilerParams(
            dimension_semantics=("parallel",),
            vmem_limit_bytes=_VMEM_LIMIT,
        ),
    )(x0p, yplanes, w9a, b3a, w1a, b1a, wct1, bct1, wc2, bc2,
      w9b, b3b, w1b, b1b, wct3, bct3, wct4, bct4)


def _planes_to_nchw(planes, B):
    P = planes.shape[3]
    y = jnp.transpose(planes, (0, 5, 3, 1, 4, 2))
    return y.reshape(B, 3, 8 * P, 8 * P)


def _nchw_to_padded_nhwc(x_nchw):
    x = jnp.transpose(x_nchw, (0, 2, 3, 1))
    x = jnp.pad(x, ((0, 0), (1, 1), (1, 1), (0, 0)))
    return x.astype(jnp.bfloat16)


def _nchw_to_planes(x_nchw):
    B, C, H, _ = x_nchw.shape
    t = jnp.transpose(x_nchw, (0, 2, 3, 1))
    t = t.reshape(B, H // 2, 2, H // 2, 2, C)
    t = jnp.transpose(t, (0, 2, 4, 1, 3, 5)).reshape(B, 4, H // 2, H // 2, C)
    t = jnp.pad(t, ((0, 0), (0, 0), (1, 1), (1, 1), (0, 0)))
    return t.astype(jnp.bfloat16)


def _pack_w9(w3):
    return jnp.concatenate(
        [w3[:, ky * 3 + kx] for kx in range(3) for ky in range(3)], axis=1)


def _pack_cat_w(wa, wb):
    parts = []
    for kx in range(3):
        for ky in range(3):
            t = ky * 3 + kx
            parts.append(wa[t])
            parts.append(wb[t])
    return jnp.concatenate(parts, axis=0)


def _pack_ct_w(wpar):
    C, Cop = wpar.shape[-2], wpar.shape[-1]
    z = jnp.zeros((C, Cop), wpar.dtype)
    rows = []
    for py in range(2):
        kblocks = []
        for c in range(3):
            for dy in range(2):
                left = wpar[2 * py, 2 * dy + c] if c <= 1 else z
                right = wpar[2 * py + 1, 2 * dy + c - 1] if c >= 1 else z
                kblocks.append(jnp.concatenate([left, right], axis=1))
        rows.append(jnp.concatenate(kblocks, axis=0))
    return jnp.stack(rows)


def _pack_ct_b(b):
    return jnp.concatenate([b, b], axis=1)


def kernel(x0, x1, r12_w3, r12_b3, r12_w1, r12_b1,
           r34_w3, r34_b3, r34_w1, r34_b1,
           ct1_w, ct1_b, ct3_w, ct3_b, ct4_w, ct4_b,
           c2_wa, c2_wb, c2_b):
    B = x0.shape[0]
    xp = _nchw_to_padded_nhwc(x0)
    yplanes = _nchw_to_planes(x1)
    packed = (
        _pack_w9(r12_w3), r12_b3, r12_w1, r12_b1,
        _pack_ct_w(ct1_w), _pack_ct_b(ct1_b),
        _pack_cat_w(c2_wa, c2_wb), c2_b,
        _pack_w9(r34_w3), r34_b3, r34_w1, r34_b1,
        _pack_ct_w(ct3_w), _pack_ct_b(ct3_b),
        _pack_ct_w(ct4_w), _pack_ct_b(ct4_b),
    )
    out = _decoder(xp, yplanes, packed)
    return _planes_to_nchw(out, B)
